# Initial kernel scaffold; baseline (speedup 1.0000x reference)
import jax
import jax.numpy as jnp
from jax.experimental import pallas as pl


def kernel(data, xs, W_conv1, b_conv1, W_att1, b_att1, W_q, b_q, W_k, b_k, W_v, b_v, gamma1, W_s1, b_s1):
    raise NotImplementedError("write your pallas kernel here")


# speedup vs baseline: 11.4055x; 11.4055x over previous
import functools

import jax
import jax.numpy as jnp
from jax import lax
from jax.experimental import pallas as pl
from jax.experimental.pallas import tpu as pltpu
from jax.experimental.pallas import tpu_sc as plsc

B, CIN, N, K = 8, 4, 2000, 20
C = 64
CH = 128
CP = 128
NEG = -3.0e38


def _conv1_body(d_ref, w_ref, b_ref, o_ref):
    d = d_ref[0]
    w = w_ref[...]
    x = lax.dot_general(d, w, (((0,), (1,)), ((), ())),
                        preferred_element_type=jnp.float32)
    o_ref[0] = x + b_ref[...]


def _conv1(d3, w, b2):
    return pl.pallas_call(
        _conv1_body,
        grid=(B,),
        in_specs=[
            pl.BlockSpec((1, CIN, N), lambda b: (b, 0, 0)),
            pl.BlockSpec((C, CIN), lambda b: (0, 0)),
            pl.BlockSpec((1, C), lambda b: (0, 0)),
        ],
        out_specs=pl.BlockSpec((1, N, C), lambda b: (b, 0, 0)),
        out_shape=jax.ShapeDtypeStruct((B, N, C), jnp.float32),
    )(d3, w, b2)


R1 = 400
T1 = N // R1


def _knn_body(xr_ref, xf_ref, o_ref):
    xr = xr_ref[0]
    xf = xf_ref[0]
    xxr = jnp.sum(xr * xr, axis=1, keepdims=True)
    xxa = jnp.sum(xf * xf, axis=0, keepdims=True)
    p = 2.0 * jnp.dot(xr, xf, preferred_element_type=jnp.float32) - xxr - xxa
    col = lax.broadcasted_iota(jnp.int32, (R1, N), 1)
    cols = []
    for _ in range(K):
        m = jnp.max(p, axis=1, keepdims=True)
        j = jnp.min(jnp.where(p >= m, col, N), axis=1, keepdims=True)
        cols.append(j)
        p = jnp.where(col == j, NEG, p)
    o_ref[0] = jnp.concatenate(cols, axis=1)


def _knn(xT, xTT):
    return pl.pallas_call(
        _knn_body,
        grid=(B, T1),
        in_specs=[
            pl.BlockSpec((1, R1, C), lambda b, t: (b, t, 0)),
            pl.BlockSpec((1, C, N), lambda b, t: (b, 0, 0)),
        ],
        out_specs=pl.BlockSpec((1, R1, K), lambda b, t: (b, t, 0)),
        out_shape=jax.ShapeDtypeStruct((B, N, K), jnp.int32),
    )(xT, xTT)


NW = 32
ROWS_W = (B * N * K) // NW
CHUNK = 80
NCH = ROWS_W // CHUNK


def _sc_gather(gidx, table):
    mesh = plsc.VectorSubcoreMesh(core_axis_name="c", subcore_axis_name="s")

    @functools.partial(
        pl.kernel,
        mesh=mesh,
        out_type=jax.ShapeDtypeStruct((B * N * K, CP), jnp.float32),
        scratch_types=[
            pltpu.VMEM((ROWS_W,), jnp.int32),
            pltpu.VMEM((CHUNK, CP), jnp.float32),
            pltpu.SemaphoreType.DMA,
        ],
    )
    def k(gidx_hbm, table_hbm, out_hbm, idx_v, rows_v, sem):
        wid = lax.axis_index("s") * 2 + lax.axis_index("c")
        base = wid * ROWS_W
        pltpu.sync_copy(gidx_hbm.at[pl.ds(base, ROWS_W)], idx_v)

        def body(c, carry):
            sl = idx_v.at[pl.ds(c * CHUNK, CHUNK)]
            pltpu.async_copy(table_hbm.at[sl], rows_v, sem).wait()
            pltpu.sync_copy(rows_v, out_hbm.at[pl.ds(base + c * CHUNK, CHUNK)])
            return carry

        lax.fori_loop(0, NCH, body, 0)

    return k(gidx, table)


T2 = K // 2


def _stats_body(g_ref, x_ref, o_ref):
    t = pl.program_id(1)
    x = x_ref[0]
    g = g_ref[0][:, :, :C]
    d = x[None] - g
    s1 = jnp.sum(d, axis=(0, 1))[None, :]
    s2 = jnp.sum(d * d, axis=(0, 1))[None, :]

    @pl.when(t == 0)
    def _():
        sx = jnp.sum(x, axis=0, keepdims=True)
        sxx = jnp.sum(x * x, axis=0, keepdims=True)
        o_ref[0] = jnp.concatenate(
            [sx, sxx, jnp.zeros((2, C), jnp.float32)], axis=0)

    acc = o_ref[0]
    o_ref[0] = acc + jnp.concatenate(
        [jnp.zeros((2, C), jnp.float32), s1, s2], axis=0)


def _stats(G, xT):
    return pl.pallas_call(
        _stats_body,
        grid=(B, T2),
        in_specs=[
            pl.BlockSpec((1, 2, N, CP), lambda b, t: (b, t, 0, 0)),
            pl.BlockSpec((1, N, C), lambda b, t: (b, 0, 0)),
        ],
        out_specs=pl.BlockSpec((1, 4, C), lambda b, t: (b, 0, 0)),
        out_shape=jax.ShapeDtypeStruct((B, 4, C), jnp.float32),
    )(G, xT)


def _mk_scales(sa, sb):
    n1 = float(N)
    nk = float(N * K)
    m1 = sa[:, 0, :] / n1
    v1 = sa[:, 1, :] / n1 - m1 * m1
    m2 = sa[:, 2, :] / nk
    v2 = sa[:, 3, :] / nk - m2 * m2
    V1 = jnp.mean(v1 / (v1 + 1e-3), axis=0, keepdims=True)
    V2 = jnp.mean(v2 / (v2 + 1e-3), axis=0, keepdims=True)
    i2A = lax.rsqrt(V1 + 1e-5)
    i2B = lax.rsqrt(V2 + 1e-5)
    mA = sb[0:1, :] / n1
    vA = sb[1:2, :] / n1 - mA * mA
    mB = sb[2:3, :] / nk
    vB = sb[3:4, :] / nk - mB * mB
    sAv = lax.rsqrt(vA + 1e-3) * i2A
    sBv = lax.rsqrt(vB + 1e-3) * i2B
    return mA, sAv, mB, sBv


R3 = 400
T3 = N // R3


def _att_body(x_ref, g_ref, sa_ref, sb_ref, wA_ref, wB_ref, b1_ref,
              wq_ref, bq_ref, wk_ref, bk_ref, wv_ref, bv_ref, gam_ref, o_ref):
    xr = x_ref[0]
    mA, sAv, mB, sBv = _mk_scales(sa_ref[...], sb_ref[0])
    aA = jnp.maximum((xr - mA) * sAv, 0.0)
    xlA = jnp.dot(aA, wA_ref[...], preferred_element_type=jnp.float32) \
        + b1_ref[...]
    es = []
    vs = []
    M = jnp.full((R3, C), NEG, jnp.float32)
    for k in range(K):
        dB = xr - g_ref[0, k][:, :C]
        aB = jnp.maximum((dB - mB) * sBv, 0.0)
        xl = xlA + jnp.dot(aB, wB_ref[...], preferred_element_type=jnp.float32)
        q = jnp.dot(xl, wq_ref[...], preferred_element_type=jnp.float32) \
            + bq_ref[...]
        kk = jnp.dot(xl, wk_ref[...], preferred_element_type=jnp.float32) \
            + bk_ref[...]
        v = jnp.dot(xl, wv_ref[...], preferred_element_type=jnp.float32) \
            + bv_ref[...]
        e = q * kk
        es.append(e)
        vs.append(v)
        M = jnp.maximum(M, e)
    ssum = jnp.zeros((R3, C), jnp.float32)
    osum = jnp.zeros((R3, C), jnp.float32)
    for k in range(K):
        pexp = jnp.exp(es[k] - M)
        ssum = ssum + pexp
        osum = osum + pexp * vs[k]
    o_ref[0] = xr + gam_ref[...] * (osum / ssum)


def _att(xT, G, st, wA, wB, b1, wq, bq, wk, bk, wv, bv, gam):
    wspec = lambda shape: pl.BlockSpec(shape, lambda b, t: (0, 0))
    return pl.pallas_call(
        _att_body,
        grid=(B, T3),
        in_specs=[
            pl.BlockSpec((1, R3, C), lambda b, t: (b, t, 0)),
            pl.BlockSpec((1, K, R3, CP), lambda b, t: (b, 0, t, 0)),
            pl.BlockSpec((B, 4, C), lambda b, t: (0, 0, 0)),
            pl.BlockSpec((1, 4, C), lambda b, t: (b, 0, 0)),
            wspec((C, C)), wspec((C, C)), wspec((1, C)),
            wspec((C, C)), wspec((1, C)),
            wspec((C, C)), wspec((1, C)),
            wspec((C, C)), wspec((1, C)),
            wspec((1, 1)),
        ],
        out_specs=pl.BlockSpec((1, R3, C), lambda b, t: (b, t, 0)),
        out_shape=jax.ShapeDtypeStruct((B, N, C), jnp.float32),
    )(xT, G, st, st, wA, wB, b1, wq, bq, wk, bk, wv, bv, gam)


def _fin_body(x_ref, g_ref, sa_ref, sb_ref, wA_ref, wB_ref, bs_ref, o_ref):
    xr = x_ref[0]
    mA, sAv, mB, sBv = _mk_scales(sa_ref[...], sb_ref[0])
    aA = jnp.maximum((xr - mA) * sAv, 0.0)
    yA = jnp.dot(aA, wA_ref[...], preferred_element_type=jnp.float32) \
        + bs_ref[...]
    out = jnp.full((R3, CH), NEG, jnp.float32)
    for k in range(K):
        dB = xr - g_ref[0, k][:, :C]
        aB = jnp.maximum((dB - mB) * sBv, 0.0)
        y = yA + jnp.dot(aB, wB_ref[...], preferred_element_type=jnp.float32)
        out = jnp.maximum(out, y)
    o_ref[0] = out


def _fin(xT, G, st, wA, wB, bs):
    wspec = lambda shape: pl.BlockSpec(shape, lambda b, t: (0, 0))
    return pl.pallas_call(
        _fin_body,
        grid=(B, T3),
        in_specs=[
            pl.BlockSpec((1, R3, C), lambda b, t: (b, t, 0)),
            pl.BlockSpec((1, K, R3, CP), lambda b, t: (b, 0, t, 0)),
            pl.BlockSpec((B, 4, C), lambda b, t: (0, 0, 0)),
            pl.BlockSpec((1, 4, C), lambda b, t: (b, 0, 0)),
            wspec((C, CH)), wspec((C, CH)), wspec((1, CH)),
        ],
        out_specs=pl.BlockSpec((1, R3, CH), lambda b, t: (b, t, 0)),
        out_shape=jax.ShapeDtypeStruct((B, N, CH), jnp.float32),
    )(xT, G, st, st, wA, wB, bs)


def _graph_gather(xT, idx):
    off = (jnp.arange(B, dtype=jnp.int32) * N)[:, None, None]
    gidx = (jnp.transpose(idx, (0, 2, 1)) + off).reshape(-1)
    table = jnp.pad(xT.reshape(B * N, C), ((0, 0), (0, CP - C)))
    G = _sc_gather(gidx, table)
    return G.reshape(B, K, N, CP)


def kernel(data, xs, W_conv1, b_conv1, W_att1, b_att1, W_q, b_q,
           W_k, b_k, W_v, b_v, gamma1, W_s1, b_s1):
    d3 = data[:, :, :, 0]
    xT1 = _conv1(d3, W_conv1, b_conv1.reshape(1, C))
    idx1 = _knn(xT1, jnp.transpose(xT1, (0, 2, 1)))
    G1 = _graph_gather(xT1, idx1)
    st1 = _stats(G1, xT1)
    xT2 = _att(
        xT1, G1, st1,
        W_att1[:, :C].T, W_att1[:, C:].T, b_att1.reshape(1, C),
        W_q.T, b_q.reshape(1, C),
        W_k.T, b_k.reshape(1, C),
        W_v.T, b_v.reshape(1, C),
        gamma1.reshape(1, 1),
    )
    idx2 = _knn(xT2, jnp.transpose(xT2, (0, 2, 1)))
    G2 = _graph_gather(xT2, idx2)
    st2 = _stats(G2, xT2)
    y = _fin(xT2, G2, st2, W_s1[:, :C].T, W_s1[:, C:].T, b_s1.reshape(1, CH))
    return jnp.transpose(y, (0, 2, 1))
